# baseline (speedup 1.0000x reference)
import functools
import math

import jax
import jax.numpy as jnp
from jax import lax
from jax.experimental import pallas as pl
from jax.experimental.pallas import tpu as pltpu, tpu_sc as plsc

_A = 73856093.0
_B = 19349663.0
_C = 83492791.0
_V = 0.0625
_TS = 2097152.0
_INV_DIAG = 1.0 / (math.sqrt(3.0) * _V)

_NPTS = 2048 * 256
_R = _NPTS // 128
_RB = 64
_TBLK = 8
_NC, _NS = 2, 16
_NW = _NC * _NS
_PW = _NPTS // _NW
_P = 256
_NCH = _PW // _P
_G = _P // 16
_NIDX = 8 * _P

_CORNERS = ((0, 0, 0), (1, 0, 0), (1, 1, 0), (0, 1, 0),
            (0, 0, 1), (1, 0, 1), (1, 1, 1), (0, 1, 1))


def _hash_tc_body(cx_ref, cy_ref, cz_ref, idx_ref, w_ref):
    x = cx_ref[...]
    y = cy_ref[...]
    z = cz_ref[...]
    bx = jnp.floor(x * 16.0) * _V
    by = jnp.floor(y * 16.0) * _V
    bz = jnp.floor(z * 16.0) * _V
    xa = (bx * _A, (bx + _V) * _A)
    yb = (by * _B, (by + _V) * _B)
    zc = (bz * _C, (bz + _V) * _C)
    fx = (x - bx, x - (bx + _V))
    fy = (y - by, y - (by + _V))
    fz = (z - bz, z - (bz + _V))
    sx = (fx[0] * fx[0], fx[1] * fx[1])
    sy = (fy[0] * fy[0], fy[1] * fy[1])
    sz = (fz[0] * fz[0], fz[1] * fz[1])
    xy = {}
    sxy = {}
    for a in (0, 1):
        for b in (0, 1):
            xy[(a, b)] = xa[a] + yb[b]
            sxy[(a, b)] = sx[a] + sy[b]
    for c, (ox, oy, oz) in enumerate(_CORNERS):
        h = xy[(ox, oy)] + zc[oz]
        m = jnp.mod(h, _TS)
        hidx = jnp.minimum(jnp.maximum(m.astype(jnp.int32), 0), 2097151)
        idx_ref[:, c, :] = hidx
        s = sxy[(ox, oy)] + sz[oz]
        w_ref[:, c, :] = jnp.sqrt(s) * _INV_DIAG


_hash_tc = pl.pallas_call(
    _hash_tc_body,
    grid=(_R // _RB,),
    in_specs=[pl.BlockSpec((_RB, 128), lambda i: (i, 0))] * 3,
    out_specs=[pl.BlockSpec((_RB, 8, 128), lambda i: (i, 0, 0))] * 2,
    out_shape=[jax.ShapeDtypeStruct((_R, 8, 128), jnp.int32),
               jax.ShapeDtypeStruct((_R, 8, 128), jnp.float32)],
)


_MESH = plsc.VectorSubcoreMesh(
    core_axis_name="c", subcore_axis_name="s",
    num_cores=_NC, num_subcores=_NS)

_NJ = 2097152 // 128
_JB = 8
_JPW = _NJ // _NW
_LITER = _JPW // _JB


@functools.partial(
    pl.kernel,
    out_type=jax.ShapeDtypeStruct((2097152, 16), jnp.float32),
    mesh=_MESH,
    compiler_params=pltpu.CompilerParams(
        needs_layout_passes=False, use_tc_tiling_on_sc=False),
    scratch_types=[
        [pltpu.VMEM((2, _JB, 8, 128), jnp.float32) for _ in range(2)],
        [pltpu.VMEM((_JB * 128, 17), jnp.float32) for _ in range(2)],
        [pltpu.SemaphoreType.DMA for _ in range(2)],
        [pltpu.SemaphoreType.DMA for _ in range(2)],
    ],
)
def _lin_sc(tp_hbm, out_hbm, slab_v, outb_v, sem_s, sem_t):
    wid = lax.axis_index("s") * _NC + lax.axis_index("c")
    j0 = wid * _JPW

    def start_slab(i, b):
        jg = j0 + i * _JB
        pltpu.async_copy(tp_hbm.at[0, pl.ds(jg, _JB)], slab_v[b].at[0],
                         sem_s[b])
        pltpu.async_copy(tp_hbm.at[1, pl.ds(jg, _JB)], slab_v[b].at[1],
                         sem_s[b])

    def wait_slab(b):
        pltpu.make_async_copy(
            tp_hbm.at[0, pl.ds(0, _JB)], slab_v[b].at[0], sem_s[b]).wait()
        pltpu.make_async_copy(
            tp_hbm.at[1, pl.ds(0, _JB)], slab_v[b].at[1], sem_s[b]).wait()

    def wait_tstore(b):
        pltpu.make_async_copy(
            out_hbm.at[pl.ds(0, _JB * 128)],
            outb_v[b].at[:, pl.ds(0, 16)], sem_t[b]).wait()

    def compute(i, b):
        iota = lax.iota(jnp.int32, 16)

        def jl_body(jl, c2):
            for l8 in range(8):
                rvec = iota + (jl * 128 + l8 * 16)
                for d in range(16):
                    xv = slab_v[b][d // 8, jl, d % 8, pl.ds(l8 * 16, 16)]
                    plsc.store_scatter(
                        outb_v[b], [rvec, jnp.full((16,), d, jnp.int32)], xv)
            return c2

        lax.fori_loop(0, _JB, jl_body, 0)

        @pl.when(i + 2 < _LITER)
        def _():
            start_slab(i + 2, b)

        pltpu.async_copy(outb_v[b].at[:, pl.ds(0, 16)],
                         out_hbm.at[pl.ds((j0 + i * _JB) * 128, _JB * 128)],
                         sem_t[b])

    start_slab(0, 0)
    start_slab(1, 1)

    def pair(i2, carry):
        for b in range(2):
            i = i2 * 2 + b
            wait_slab(b)

            @pl.when(i >= 2)
            def _():
                wait_tstore(b)

            compute(i, b)
        return carry

    lax.fori_loop(0, _LITER // 2, pair, 0)
    wait_tstore(0)
    wait_tstore(1)


@functools.partial(
    pl.kernel,
    out_type=jax.ShapeDtypeStruct((2048, 2, 2, 8, 128), jnp.float32),
    mesh=_MESH,
    compiler_params=pltpu.CompilerParams(
        needs_layout_passes=False, use_tc_tiling_on_sc=False),
    scratch_types=[
        [pltpu.VMEM((_NIDX,), jnp.int32) for _ in range(4)],
        [pltpu.VMEM((_NIDX,), jnp.float32) for _ in range(4)],
        [pltpu.VMEM((_NIDX, 16), jnp.float32) for _ in range(2)],
        [pltpu.VMEM((2, 2, 8, 128), jnp.float32) for _ in range(2)],
        [pltpu.SemaphoreType.DMA for _ in range(4)],
        [pltpu.SemaphoreType.DMA for _ in range(2)],
        [pltpu.SemaphoreType.DMA for _ in range(2)],
    ],
)
def _gather_sc(idx_hbm, w_hbm, table_hbm, out_hbm,
               idx_v, w_v, rows_v, out_v, sem_iw, sem_g, sem_o):
    wid = lax.axis_index("s") * _NC + lax.axis_index("c")
    q0w = wid * (_PW * 8)
    n0 = wid * _NCH

    def start_iw(n, b4):
        q0 = q0w + n * _NIDX
        pltpu.async_copy(idx_hbm.at[pl.ds(q0, _NIDX)], idx_v[b4], sem_iw[b4])
        pltpu.async_copy(w_hbm.at[pl.ds(q0, _NIDX)], w_v[b4], sem_iw[b4])

    def wait_iw(b4):
        pltpu.make_async_copy(
            idx_hbm.at[pl.ds(0, _NIDX)], idx_v[b4], sem_iw[b4]).wait()
        pltpu.make_async_copy(
            w_hbm.at[pl.ds(0, _NIDX)], w_v[b4], sem_iw[b4]).wait()

    def fire_gathers(b4, b2):
        pltpu.async_copy(
            table_hbm.at[idx_v[b4]], rows_v[b2], sem_g[b2])

    def wait_gathers(b2):
        pltpu.make_async_copy(
            table_hbm.at[pl.ds(0, _NIDX)], rows_v[b2], sem_g[b2]).wait()

    def wait_out(b2):
        pltpu.make_async_copy(
            out_hbm.at[0], out_v[b2], sem_o[b2]).wait()

    def compute(n, b4, b2):
        def g2(g, c2):
            off = g * 16
            jj = g // 8
            col = (g % 8) * 16
            tbase = jj * 1024 + col
            wl = [w_v[b4][pl.ds(tbase + c * 128, 16)] for c in range(8)]
            tv = [lax.iota(jnp.int32, 16) + (tbase + c * 128)
                  for c in range(8)]
            for d in range(16):
                dv = jnp.full((16,), d, jnp.int32)
                acc = wl[0] * plsc.load_gather(rows_v[b2], [tv[0], dv])
                for c in range(1, 8):
                    acc = acc + wl[c] * plsc.load_gather(
                        rows_v[b2], [tv[c], dv])
                out_v[b2][d // 8, jj, d % 8, pl.ds(col, 16)] = acc
            return c2

        lax.fori_loop(0, _G, g2, 0)
        pltpu.async_copy(out_v[b2], out_hbm.at[n0 + n], sem_o[b2])

    start_iw(0, 0)
    start_iw(1, 1)

    def quad(i, carry):
        for k in range(4):
            n = i * 4 + k
            wait_iw(k)
            fire_gathers(k, k % 2)

            @pl.when(n + 2 < _NCH)
            def _():
                start_iw(n + 2, (k + 2) % 4)

            @pl.when(n > 0)
            def _():
                wait_gathers((k + 1) % 2)

                @pl.when(n >= 3)
                def _():
                    wait_out((k + 1) % 2)

                compute(n - 1, (k + 3) % 4, (k + 1) % 2)

        return carry

    lax.fori_loop(0, _NCH // 4, quad, 0)

    wait_gathers((_NCH - 1) % 2)
    wait_out((_NCH - 1) % 2)
    compute(_NCH - 1, (_NCH - 1) % 4, (_NCH - 1) % 2)
    wait_out(0)
    wait_out(1)


def kernel(coords, table):
    n, t, _ = coords.shape
    flat = coords.reshape(-1, 3)
    cx = flat[:, 0].reshape(_R, 128)
    cy = flat[:, 1].reshape(_R, 128)
    cz = flat[:, 2].reshape(_R, 128)
    idx, w = _hash_tc(cx, cy, cz)
    tp = table.reshape(16384, 128, 2, 8).transpose(2, 0, 3, 1)
    tlin = _lin_sc(tp)
    out5 = _gather_sc(idx.reshape(-1), w.reshape(-1), tlin)
    return out5.transpose(0, 2, 4, 1, 3).reshape(n, t, 16)


# speedup vs baseline: 1.1845x; 1.1845x over previous
import functools
import math

import jax
import jax.numpy as jnp
from jax import lax
from jax.experimental import pallas as pl
from jax.experimental.pallas import tpu as pltpu, tpu_sc as plsc

_A = 73856093.0
_B = 19349663.0
_C = 83492791.0
_V = 0.0625
_TS = 2097152.0
_INV_DIAG = 1.0 / (math.sqrt(3.0) * _V)

_NPTS = 2048 * 256
_R = _NPTS // 128
_RB = 64
_TBLK = 8
_NC, _NS = 2, 16
_NW = _NC * _NS
_PW = _NPTS // _NW
_P = 256
_NCH = _PW // _P
_G = _P // 16
_NIDX = 8 * _P

_CORNERS = ((0, 0, 0), (1, 0, 0), (1, 1, 0), (0, 1, 0),
            (0, 0, 1), (1, 0, 1), (1, 1, 1), (0, 1, 1))


def _hash_tc_body(cx_ref, cy_ref, cz_ref, idx_ref, w_ref):
    x = cx_ref[...]
    y = cy_ref[...]
    z = cz_ref[...]
    bx = jnp.floor(x * 16.0) * _V
    by = jnp.floor(y * 16.0) * _V
    bz = jnp.floor(z * 16.0) * _V
    xa = (bx * _A, (bx + _V) * _A)
    yb = (by * _B, (by + _V) * _B)
    zc = (bz * _C, (bz + _V) * _C)
    fx = (x - bx, x - (bx + _V))
    fy = (y - by, y - (by + _V))
    fz = (z - bz, z - (bz + _V))
    sx = (fx[0] * fx[0], fx[1] * fx[1])
    sy = (fy[0] * fy[0], fy[1] * fy[1])
    sz = (fz[0] * fz[0], fz[1] * fz[1])
    xy = {}
    sxy = {}
    for a in (0, 1):
        for b in (0, 1):
            xy[(a, b)] = xa[a] + yb[b]
            sxy[(a, b)] = sx[a] + sy[b]
    for c, (ox, oy, oz) in enumerate(_CORNERS):
        h = xy[(ox, oy)] + zc[oz]
        m = jnp.mod(h, _TS)
        hidx = jnp.minimum(jnp.maximum(m.astype(jnp.int32), 0), 2097151)
        idx_ref[:, c, :] = hidx
        s = sxy[(ox, oy)] + sz[oz]
        w_ref[:, c, :] = jnp.sqrt(s) * _INV_DIAG


_hash_tc = pl.pallas_call(
    _hash_tc_body,
    grid=(_R // _RB,),
    in_specs=[pl.BlockSpec((_RB, 128), lambda i: (i, 0))] * 3,
    out_specs=[pl.BlockSpec((_RB, 8, 128), lambda i: (i, 0, 0))] * 2,
    out_shape=[jax.ShapeDtypeStruct((_R, 8, 128), jnp.int32),
               jax.ShapeDtypeStruct((_R, 8, 128), jnp.float32)],
)


_MESH = plsc.VectorSubcoreMesh(
    core_axis_name="c", subcore_axis_name="s",
    num_cores=_NC, num_subcores=_NS)

_NJ = 2097152 // 128
_JB = 8
_JPW = _NJ // _NW
_LITER = _JPW // _JB


@functools.partial(
    pl.kernel,
    out_type=jax.ShapeDtypeStruct((2097152, 16), jnp.float32),
    mesh=_MESH,
    compiler_params=pltpu.CompilerParams(
        needs_layout_passes=False, use_tc_tiling_on_sc=False),
    scratch_types=[
        [pltpu.VMEM((2, _JB, 8, 128), jnp.float32) for _ in range(2)],
        [pltpu.VMEM((_JB * 128, 16), jnp.float32) for _ in range(2)],
        [pltpu.SemaphoreType.DMA for _ in range(2)],
        [pltpu.SemaphoreType.DMA for _ in range(2)],
    ],
)
def _lin_sc(tp_hbm, out_hbm, slab_v, outb_v, sem_s, sem_t):
    wid = lax.axis_index("s") * _NC + lax.axis_index("c")
    j0 = wid * _JPW

    def start_slab(i, b):
        jg = j0 + i * _JB
        pltpu.async_copy(tp_hbm.at[0, pl.ds(jg, _JB)], slab_v[b].at[0],
                         sem_s[b])
        pltpu.async_copy(tp_hbm.at[1, pl.ds(jg, _JB)], slab_v[b].at[1],
                         sem_s[b])

    def wait_slab(b):
        pltpu.make_async_copy(
            tp_hbm.at[0, pl.ds(0, _JB)], slab_v[b].at[0], sem_s[b]).wait()
        pltpu.make_async_copy(
            tp_hbm.at[1, pl.ds(0, _JB)], slab_v[b].at[1], sem_s[b]).wait()

    def wait_tstore(b):
        pltpu.make_async_copy(
            out_hbm.at[pl.ds(0, _JB * 128)], outb_v[b], sem_t[b]).wait()

    def compute(i, b):
        iota = lax.iota(jnp.int32, 16)
        dmod = [(iota + k) & 15 for k in range(16)]

        def jl_body(jl, c2):
            jlv = iota * 0 + jl
            for l8 in range(8):
                lvec = iota + (l8 * 16)
                rvec = iota + (jl * 128 + l8 * 16)
                for k in range(16):
                    diag = plsc.load_gather(
                        slab_v[b], [dmod[k] >> 3, jlv, dmod[k] & 7, lvec])
                    plsc.store_scatter(outb_v[b], [rvec, dmod[k]], diag)
            return c2

        lax.fori_loop(0, _JB, jl_body, 0)

        @pl.when(i + 2 < _LITER)
        def _():
            start_slab(i + 2, b)

        pltpu.async_copy(outb_v[b],
                         out_hbm.at[pl.ds((j0 + i * _JB) * 128, _JB * 128)],
                         sem_t[b])

    start_slab(0, 0)
    start_slab(1, 1)

    def pair(i2, carry):
        for b in range(2):
            i = i2 * 2 + b
            wait_slab(b)

            @pl.when(i >= 2)
            def _():
                wait_tstore(b)

            compute(i, b)
        return carry

    lax.fori_loop(0, _LITER // 2, pair, 0)
    wait_tstore(0)
    wait_tstore(1)


@functools.partial(
    pl.kernel,
    out_type=jax.ShapeDtypeStruct((2048, 2, 2, 8, 128), jnp.float32),
    mesh=_MESH,
    compiler_params=pltpu.CompilerParams(
        needs_layout_passes=False, use_tc_tiling_on_sc=False),
    scratch_types=[
        [pltpu.VMEM((_NIDX,), jnp.int32) for _ in range(4)],
        [pltpu.VMEM((_NIDX,), jnp.float32) for _ in range(4)],
        [pltpu.VMEM((_NIDX, 16), jnp.float32) for _ in range(2)],
        [pltpu.VMEM((2, 2, 8, 128), jnp.float32) for _ in range(2)],
        [pltpu.SemaphoreType.DMA for _ in range(4)],
        [pltpu.SemaphoreType.DMA for _ in range(2)],
        [pltpu.SemaphoreType.DMA for _ in range(2)],
    ],
)
def _gather_sc(idx_hbm, w_hbm, table_hbm, out_hbm,
               idx_v, w_v, rows_v, out_v, sem_iw, sem_g, sem_o):
    wid = lax.axis_index("s") * _NC + lax.axis_index("c")
    q0w = wid * (_PW * 8)
    n0 = wid * _NCH

    def start_iw(n, b4):
        q0 = q0w + n * _NIDX
        pltpu.async_copy(idx_hbm.at[pl.ds(q0, _NIDX)], idx_v[b4], sem_iw[b4])
        pltpu.async_copy(w_hbm.at[pl.ds(q0, _NIDX)], w_v[b4], sem_iw[b4])

    def wait_iw(b4):
        pltpu.make_async_copy(
            idx_hbm.at[pl.ds(0, _NIDX)], idx_v[b4], sem_iw[b4]).wait()
        pltpu.make_async_copy(
            w_hbm.at[pl.ds(0, _NIDX)], w_v[b4], sem_iw[b4]).wait()

    def fire_gathers(b4, b2):
        pltpu.async_copy(
            table_hbm.at[idx_v[b4]], rows_v[b2], sem_g[b2])

    def wait_gathers(b2):
        pltpu.make_async_copy(
            table_hbm.at[pl.ds(0, _NIDX)], rows_v[b2], sem_g[b2]).wait()

    def wait_out(b2):
        pltpu.make_async_copy(
            out_hbm.at[0], out_v[b2], sem_o[b2]).wait()

    def compute(n, b4, b2):
        def g2(g, c2):
            off = g * 16
            jj = g // 8
            col = (g % 8) * 16
            tbase = jj * 1024 + col
            wl = [w_v[b4][pl.ds(tbase + c * 128, 16)] for c in range(8)]
            tv = [lax.iota(jnp.int32, 16) + (tbase + c * 128)
                  for c in range(8)]
            for d in range(16):
                dv = jnp.full((16,), d, jnp.int32)
                acc = wl[0] * plsc.load_gather(rows_v[b2], [tv[0], dv])
                for c in range(1, 8):
                    acc = acc + wl[c] * plsc.load_gather(
                        rows_v[b2], [tv[c], dv])
                out_v[b2][d // 8, jj, d % 8, pl.ds(col, 16)] = acc
            return c2

        lax.fori_loop(0, _G, g2, 0)
        pltpu.async_copy(out_v[b2], out_hbm.at[n0 + n], sem_o[b2])

    start_iw(0, 0)
    start_iw(1, 1)

    def quad(i, carry):
        for k in range(4):
            n = i * 4 + k
            wait_iw(k)
            fire_gathers(k, k % 2)

            @pl.when(n + 2 < _NCH)
            def _():
                start_iw(n + 2, (k + 2) % 4)

            @pl.when(n > 0)
            def _():
                wait_gathers((k + 1) % 2)

                @pl.when(n >= 3)
                def _():
                    wait_out((k + 1) % 2)

                compute(n - 1, (k + 3) % 4, (k + 1) % 2)

        return carry

    lax.fori_loop(0, _NCH // 4, quad, 0)

    wait_gathers((_NCH - 1) % 2)
    wait_out((_NCH - 1) % 2)
    compute(_NCH - 1, (_NCH - 1) % 4, (_NCH - 1) % 2)
    wait_out(0)
    wait_out(1)


def kernel(coords, table):
    n, t, _ = coords.shape
    flat = coords.reshape(-1, 3)
    cx = flat[:, 0].reshape(_R, 128)
    cy = flat[:, 1].reshape(_R, 128)
    cz = flat[:, 2].reshape(_R, 128)
    idx, w = _hash_tc(cx, cy, cz)
    tp = table.reshape(16384, 128, 2, 8).transpose(2, 0, 3, 1)
    tlin = _lin_sc(tp)
    out5 = _gather_sc(idx.reshape(-1), w.reshape(-1), tlin)
    return out5.transpose(0, 2, 4, 1, 3).reshape(n, t, 16)


# speedup vs baseline: 1.3365x; 1.1283x over previous
import functools
import math

import jax
import jax.numpy as jnp
from jax import lax
from jax.experimental import pallas as pl
from jax.experimental.pallas import tpu as pltpu, tpu_sc as plsc

_A = 73856093.0
_B = 19349663.0
_C = 83492791.0
_V = 0.0625
_TS = 2097152.0
_INV_DIAG = 1.0 / (math.sqrt(3.0) * _V)

_NPTS = 2048 * 256
_R = _NPTS // 128
_RB = 64
_TBLK = 8
_NC, _NS = 2, 16
_NW = _NC * _NS
_PW = _NPTS // _NW
_P = 256
_NCH = _PW // _P
_G = _P // 16
_NIDX = 8 * _P

_CORNERS = ((0, 0, 0), (1, 0, 0), (1, 1, 0), (0, 1, 0),
            (0, 0, 1), (1, 0, 1), (1, 1, 1), (0, 1, 1))


def _hash_tc_body(cx_ref, cy_ref, cz_ref, idx_ref, w_ref):
    x = cx_ref[...]
    y = cy_ref[...]
    z = cz_ref[...]
    bx = jnp.floor(x * 16.0) * _V
    by = jnp.floor(y * 16.0) * _V
    bz = jnp.floor(z * 16.0) * _V
    xa = (bx * _A, (bx + _V) * _A)
    yb = (by * _B, (by + _V) * _B)
    zc = (bz * _C, (bz + _V) * _C)
    fx = (x - bx, x - (bx + _V))
    fy = (y - by, y - (by + _V))
    fz = (z - bz, z - (bz + _V))
    sx = (fx[0] * fx[0], fx[1] * fx[1])
    sy = (fy[0] * fy[0], fy[1] * fy[1])
    sz = (fz[0] * fz[0], fz[1] * fz[1])
    xy = {}
    sxy = {}
    for a in (0, 1):
        for b in (0, 1):
            xy[(a, b)] = xa[a] + yb[b]
            sxy[(a, b)] = sx[a] + sy[b]
    for c, (ox, oy, oz) in enumerate(_CORNERS):
        h = xy[(ox, oy)] + zc[oz]
        m = jnp.mod(h, _TS)
        hidx = jnp.minimum(jnp.maximum(m.astype(jnp.int32), 0), 2097151)
        idx_ref[:, c, :] = hidx
        s = sxy[(ox, oy)] + sz[oz]
        w_ref[:, c, :] = jnp.sqrt(s) * _INV_DIAG


_hash_tc = pl.pallas_call(
    _hash_tc_body,
    grid=(_R // _RB,),
    in_specs=[pl.BlockSpec((_RB, 128), lambda i: (i, 0))] * 3,
    out_specs=[pl.BlockSpec((_RB, 8, 128), lambda i: (i, 0, 0))] * 2,
    out_shape=[jax.ShapeDtypeStruct((_R, 8, 128), jnp.int32),
               jax.ShapeDtypeStruct((_R, 8, 128), jnp.float32)],
)


_MESH = plsc.VectorSubcoreMesh(
    core_axis_name="c", subcore_axis_name="s",
    num_cores=_NC, num_subcores=_NS)

_NJ = 2097152 // 128
_JB = 8
_JPW = _NJ // _NW
_LITER = _JPW // _JB


@functools.partial(
    pl.kernel,
    out_type=jax.ShapeDtypeStruct((2097152, 16), jnp.float32),
    mesh=_MESH,
    compiler_params=pltpu.CompilerParams(
        needs_layout_passes=False, use_tc_tiling_on_sc=False),
    scratch_types=[
        [pltpu.VMEM((2, _JB, 8, 128), jnp.float32) for _ in range(2)],
        [pltpu.VMEM((_JB * 128, 16), jnp.float32) for _ in range(2)],
        [pltpu.SemaphoreType.DMA for _ in range(2)],
        [pltpu.SemaphoreType.DMA for _ in range(2)],
    ],
)
def _lin_sc(tp_hbm, out_hbm, slab_v, outb_v, sem_s, sem_t):
    wid = lax.axis_index("s") * _NC + lax.axis_index("c")
    j0 = wid * _JPW

    def start_slab(i, b):
        jg = j0 + i * _JB
        pltpu.async_copy(tp_hbm.at[0, pl.ds(jg, _JB)], slab_v[b].at[0],
                         sem_s[b])
        pltpu.async_copy(tp_hbm.at[1, pl.ds(jg, _JB)], slab_v[b].at[1],
                         sem_s[b])

    def wait_slab(b):
        pltpu.make_async_copy(
            tp_hbm.at[0, pl.ds(0, _JB)], slab_v[b].at[0], sem_s[b]).wait()
        pltpu.make_async_copy(
            tp_hbm.at[1, pl.ds(0, _JB)], slab_v[b].at[1], sem_s[b]).wait()

    def wait_tstore(b):
        pltpu.make_async_copy(
            out_hbm.at[pl.ds(0, _JB * 128)], outb_v[b], sem_t[b]).wait()

    def compute(i, b):
        iota = lax.iota(jnp.int32, 16)

        def jl_body(jl, c2):
            for l8 in range(8):
                rvec = iota + (jl * 128 + l8 * 16)
                for d in range(16):
                    xv = slab_v[b][d // 8, jl, d % 8, pl.ds(l8 * 16, 16)]
                    plsc.store_scatter(
                        outb_v[b], [rvec, jnp.full((16,), d, jnp.int32)], xv)
            return c2

        lax.fori_loop(0, _JB, jl_body, 0)

        @pl.when(i + 2 < _LITER)
        def _():
            start_slab(i + 2, b)

        pltpu.async_copy(outb_v[b],
                         out_hbm.at[pl.ds((j0 + i * _JB) * 128, _JB * 128)],
                         sem_t[b])

    start_slab(0, 0)
    start_slab(1, 1)

    def pair(i2, carry):
        for b in range(2):
            i = i2 * 2 + b
            wait_slab(b)

            @pl.when(i >= 2)
            def _():
                wait_tstore(b)

            compute(i, b)
        return carry

    lax.fori_loop(0, _LITER // 2, pair, 0)
    wait_tstore(0)
    wait_tstore(1)


@functools.partial(
    pl.kernel,
    out_type=jax.ShapeDtypeStruct((2048, 2, 2, 8, 128), jnp.float32),
    mesh=_MESH,
    compiler_params=pltpu.CompilerParams(
        needs_layout_passes=False, use_tc_tiling_on_sc=False),
    scratch_types=[
        [pltpu.VMEM((_NIDX,), jnp.int32) for _ in range(4)],
        [pltpu.VMEM((_NIDX,), jnp.float32) for _ in range(4)],
        [pltpu.VMEM((_NIDX, 16), jnp.float32) for _ in range(2)],
        [pltpu.VMEM((2, 2, 8, 128), jnp.float32) for _ in range(2)],
        [pltpu.SemaphoreType.DMA for _ in range(4)],
        [pltpu.SemaphoreType.DMA for _ in range(2)],
        [pltpu.SemaphoreType.DMA for _ in range(2)],
    ],
)
def _gather_sc(idx_hbm, w_hbm, table_hbm, out_hbm,
               idx_v, w_v, rows_v, out_v, sem_iw, sem_g, sem_o):
    wid = lax.axis_index("s") * _NC + lax.axis_index("c")
    q0w = wid * (_PW * 8)
    n0 = wid * _NCH

    def start_iw(n, b4):
        q0 = q0w + n * _NIDX
        pltpu.async_copy(idx_hbm.at[pl.ds(q0, _NIDX)], idx_v[b4], sem_iw[b4])
        pltpu.async_copy(w_hbm.at[pl.ds(q0, _NIDX)], w_v[b4], sem_iw[b4])

    def wait_iw(b4):
        pltpu.make_async_copy(
            idx_hbm.at[pl.ds(0, _NIDX)], idx_v[b4], sem_iw[b4]).wait()
        pltpu.make_async_copy(
            w_hbm.at[pl.ds(0, _NIDX)], w_v[b4], sem_iw[b4]).wait()

    def fire_gathers(b4, b2):
        pltpu.async_copy(
            table_hbm.at[idx_v[b4]], rows_v[b2], sem_g[b2])

    def wait_gathers(b2):
        pltpu.make_async_copy(
            table_hbm.at[pl.ds(0, _NIDX)], rows_v[b2], sem_g[b2]).wait()

    def wait_out(b2):
        pltpu.make_async_copy(
            out_hbm.at[0], out_v[b2], sem_o[b2]).wait()

    def compute(n, b4, b2):
        def g2(g, c2):
            off = g * 16
            jj = g // 8
            col = (g % 8) * 16
            tbase = jj * 1024 + col
            wl = [w_v[b4][pl.ds(tbase + c * 128, 16)] for c in range(8)]
            tv = [lax.iota(jnp.int32, 16) + (tbase + c * 128)
                  for c in range(8)]
            for d in range(16):
                dv = jnp.full((16,), d, jnp.int32)
                acc = wl[0] * plsc.load_gather(rows_v[b2], [tv[0], dv])
                for c in range(1, 8):
                    acc = acc + wl[c] * plsc.load_gather(
                        rows_v[b2], [tv[c], dv])
                out_v[b2][d // 8, jj, d % 8, pl.ds(col, 16)] = acc
            return c2

        lax.fori_loop(0, _G, g2, 0)
        pltpu.async_copy(out_v[b2], out_hbm.at[n0 + n], sem_o[b2])

    start_iw(0, 0)
    start_iw(1, 1)

    def quad(i, carry):
        for k in range(4):
            n = i * 4 + k
            wait_iw(k)
            fire_gathers(k, k % 2)

            @pl.when(n + 2 < _NCH)
            def _():
                start_iw(n + 2, (k + 2) % 4)

            @pl.when(n > 0)
            def _():
                wait_gathers((k + 1) % 2)

                @pl.when(n >= 3)
                def _():
                    wait_out((k + 1) % 2)

                compute(n - 1, (k + 3) % 4, (k + 1) % 2)

        return carry

    lax.fori_loop(0, _NCH // 4, quad, 0)

    wait_gathers((_NCH - 1) % 2)
    wait_out((_NCH - 1) % 2)
    compute(_NCH - 1, (_NCH - 1) % 4, (_NCH - 1) % 2)
    wait_out(0)
    wait_out(1)


def kernel(coords, table):
    n, t, _ = coords.shape
    flat = coords.reshape(-1, 3)
    cx = flat[:, 0].reshape(_R, 128)
    cy = flat[:, 1].reshape(_R, 128)
    cz = flat[:, 2].reshape(_R, 128)
    idx, w = _hash_tc(cx, cy, cz)
    tp = table.reshape(16384, 128, 2, 8).transpose(2, 0, 3, 1)
    tlin = _lin_sc(tp)
    out5 = _gather_sc(idx.reshape(-1), w.reshape(-1), tlin)
    return out5.transpose(0, 2, 4, 1, 3).reshape(n, t, 16)
